# Initial kernel scaffold; baseline (speedup 1.0000x reference)
import jax
import jax.numpy as jnp
from jax.experimental import pallas as pl


def kernel(input):
    raise NotImplementedError("write your pallas kernel here")


# speedup vs baseline: 2.0438x; 2.0438x over previous
import functools

import jax
import jax.numpy as jnp
from jax import lax
from jax.experimental import pallas as pl
from jax.experimental.pallas import tpu as pltpu
from jax.experimental.pallas import tpu_sc as plsc

BATCH = 1024
SEQ_LEN = 2048
ALPHABET = 4
N = BATCH * SEQ_LEN

NUM_CORES = 2
NUM_SUBCORES = 16
NW = NUM_CORES * NUM_SUBCORES
PER_W = N // NW
CHUNK = 8192
NCH = PER_W // CHUNK
NBUF = 2
LANES = 16
VPC = CHUNK // LANES


def _body(in_hbm, out_hbm, in_v, out_v, in_sem, out_sem):
    wid = lax.axis_index("s") * NUM_CORES + lax.axis_index("c")
    base = wid * PER_W

    lane = lax.iota(jnp.int32, LANES)
    chan = lane & 3
    rep = [(lane >> 2) + 4 * t for t in range(4)]

    def start_in(g, b):
        return pltpu.async_copy(
            in_hbm.at[pl.ds(base + g * CHUNK, CHUNK)], in_v.at[b], in_sem)

    def compute_chunk(b):
        in_ref = in_v.at[b]

        def jbody(j, carry):
            vals16 = in_ref[pl.ds(j * LANES, LANES)]
            for t in range(4):
                vals = vals16.at[rep[t]].get(mode="promise_in_bounds")
                oh = jnp.where(vals == chan, jnp.float32(1.0), jnp.float32(0.0))
                out_v[b, pl.ds(j * (4 * LANES) + t * LANES, LANES)] = oh
            return carry

        lax.fori_loop(0, VPC, jbody, 0, unroll=2)

    def start_out(g, b):
        return pltpu.async_copy(
            out_v.at[b], out_hbm.at[pl.ds((base + g * CHUNK) * 4, CHUNK * 4)],
            out_sem)

    in_copies = [start_in(0, 0)]
    out_copies = []
    for g in range(NCH):
        b = g % NBUF
        if g + 1 < NCH:
            in_copies.append(start_in(g + 1, (g + 1) % NBUF))
        in_copies[g].wait()
        if g >= NBUF:
            out_copies[g - NBUF].wait()
        compute_chunk(b)
        out_copies.append(start_out(g, b))
    for g in range(NCH - NBUF, NCH):
        out_copies[g].wait()


@jax.jit
def _one_hot_sc(flat_in):
    mesh = plsc.VectorSubcoreMesh(
        core_axis_name="c", subcore_axis_name="s",
        num_cores=NUM_CORES, num_subcores=NUM_SUBCORES)
    return pl.kernel(
        _body,
        out_type=jax.ShapeDtypeStruct((N * 4,), jnp.float32),
        mesh=mesh,
        scratch_types=[
            pltpu.VMEM((NBUF, CHUNK), jnp.int32),
            pltpu.VMEM((NBUF, CHUNK * 4), jnp.float32),
            pltpu.SemaphoreType.DMA,
            pltpu.SemaphoreType.DMA,
        ],
    )(flat_in)


def kernel(input):
    out_flat = _one_hot_sc(input.reshape(N))
    return out_flat.reshape(BATCH, SEQ_LEN, ALPHABET)


# speedup vs baseline: 50.4564x; 24.6879x over previous
import functools

import jax
import jax.numpy as jnp
from jax import lax
from jax.experimental import pallas as pl
from jax.experimental.pallas import tpu as pltpu
from jax.experimental.pallas import tpu_sc as plsc

BATCH = 1024
SEQ_LEN = 2048
ALPHABET = 4
N = BATCH * SEQ_LEN

NUM_CORES = 2
NUM_SUBCORES = 16
NW = NUM_CORES * NUM_SUBCORES
NSLAB = BATCH // 8
SLABS_PER_W = NSLAB // NW
IN_SLAB = 8 * SEQ_LEN
OUT_HALF = 4 * SEQ_LEN * 4
LANES = 16
NT = SEQ_LEN // 128


def _body(in_hbm, out_hbm, in_v, out_v, in_sem, out_sem):
    wid = lax.axis_index("s") * NUM_CORES + lax.axis_index("c")
    slab0 = wid * SLABS_PER_W

    one = jnp.full((LANES,), 1.0, jnp.float32)
    zero = jnp.zeros((LANES,), jnp.float32)

    def start_in(i):
        return pltpu.async_copy(
            in_hbm.at[pl.ds((slab0 + i) * IN_SLAB, IN_SLAB)],
            in_v.at[i % 2], in_sem)

    def start_out(i, h, ob):
        return pltpu.async_copy(
            out_v.at[ob],
            out_hbm.at[pl.ds(((slab0 + i) * 2 + h) * OUT_HALF, OUT_HALF)],
            out_sem)

    def compute_half(ib, h, ob):
        in_ref = in_v.at[ib]
        out_ref = out_v.at[ob]

        def jbody(j, carry):
            t = j // 4
            bp = j % 4
            bin_ = (t * 8 + 4 * h + bp) * 128
            bout = bp * 8192 + t * 512
            for k in range(8):
                vals = in_ref[pl.ds(bin_ + k * LANES, LANES)]
                for c in range(ALPHABET):
                    out_ref[pl.ds(bout + c * 128 + k * LANES, LANES)] = (
                        jnp.where(vals == c, one, zero))
            return carry

        lax.fori_loop(0, 4 * NT, jbody, 0)

    in_copies = [start_in(0)]
    out_copies = []
    step = 0
    for i in range(SLABS_PER_W):
        if i + 1 < SLABS_PER_W:
            in_copies.append(start_in(i + 1))
        in_copies[i].wait()
        for h in range(2):
            ob = step % 2
            if step >= 2:
                out_copies[step - 2].wait()
            compute_half(i % 2, h, ob)
            out_copies.append(start_out(i, h, ob))
            step += 1
    out_copies[step - 2].wait()
    out_copies[step - 1].wait()


@jax.jit
def _one_hot_sc(flat_in):
    mesh = plsc.VectorSubcoreMesh(
        core_axis_name="c", subcore_axis_name="s",
        num_cores=NUM_CORES, num_subcores=NUM_SUBCORES)
    return pl.kernel(
        _body,
        out_type=jax.ShapeDtypeStruct((N * 4,), jnp.float32),
        mesh=mesh,
        scratch_types=[
            pltpu.VMEM((2, IN_SLAB), jnp.int32),
            pltpu.VMEM((2, OUT_HALF), jnp.float32),
            pltpu.SemaphoreType.DMA,
            pltpu.SemaphoreType.DMA,
        ],
    )(flat_in)


def kernel(input):
    flat_in = (input.reshape(NSLAB, 8, NT, 128)
               .transpose(0, 2, 1, 3)
               .reshape(N))
    out_flat = _one_hot_sc(flat_in)
    return (out_flat.reshape(BATCH, NT, ALPHABET, 128)
            .transpose(0, 1, 3, 2)
            .reshape(BATCH, SEQ_LEN, ALPHABET))
